# baseline (speedup 1.0000x reference)
import functools
import jax
import jax.numpy as jnp
from jax import lax
from jax.experimental import pallas as pl
from jax.experimental.pallas import tpu as pltpu
from jax.experimental.pallas import tpu_sc as plsc

_TOPK = 16
_WS = 4
_WIN = 64
_NS = 1024
_C = 384
_G4 = 96
_NTOK = _TOPK * _WS * _WS
_NB = 8


def _sel_body(z_ref, x_ref, pool_ref, rows_ref):
    for b in range(_NB):
        zmax = jnp.max(z_ref[b], axis=0, keepdims=True)
        zr = zmax.astype(jnp.bfloat16).astype(jnp.float32)
        xb = x_ref[b].astype(jnp.bfloat16)
        A = jnp.dot(pool_ref[...], xb, preferred_element_type=jnp.float32)
        wsum = jnp.sum(A * zr, axis=1, keepdims=True)
        wrow = jnp.transpose(wsum)
        iv = lax.broadcasted_iota(jnp.int32, (_WIN, _WIN), 0)
        iw = lax.broadcasted_iota(jnp.int32, (_WIN, _WIN), 1)
        beats2 = (wrow > wsum) | ((wrow == wsum) & (iw < iv))
        rank_col = jnp.sum(beats2.astype(jnp.int32), axis=1,
                           keepdims=True)
        slane = lax.broadcasted_iota(jnp.int32, (1, _WIN), 1)
        oh = (rank_col == slane).astype(jnp.float32)
        wiota = lax.broadcasted_iota(jnp.int32, (_WIN, 1), 0).astype(jnp.float32)
        invperm = lax.dot_general(wiota, oh,
                                  dimension_numbers=(((0,), (0,)), ((), ())),
                                  preferred_element_type=jnp.float32)
        idx16 = invperm[:, 0:_TOPK]
        iexp = lax.broadcasted_iota(jnp.int32, (_TOPK, _NTOK), 1) // 16
        irow = lax.broadcasted_iota(jnp.int32, (_TOPK, _NTOK), 0)
        E = (iexp == irow).astype(jnp.float32)
        idx256 = jnp.dot(idx16, E,
                         preferred_element_type=jnp.float32).astype(jnp.int32)
        col = lax.broadcasted_iota(jnp.int32, (1, _NTOK), 1)
        offs = ((col % 16) // 4) * 32 + (col % 4)
        bglob = pl.program_id(0) * _NB + b
        rows = (idx256 // 8) * 128 + (idx256 % 8) * 4 + offs + bglob * _NS
        rows_ref[pl.ds(b, 1), :] = rows


def _proj_body(xg_ref, wdT_ref, bd_ref, wuT_ref, bu_ref, out_ref):
    for b in range(_NB):
        x_g = xg_ref[b]
        t = jnp.dot(x_g.astype(jnp.bfloat16), wdT_ref[...],
                    preferred_element_type=jnp.float32) + bd_ref[...]
        zero1 = jnp.zeros((1, _G4), jnp.float32)
        zero4 = jnp.zeros((4, _G4), jnp.float32)
        tp1 = jnp.concatenate([t[1:], zero1], axis=0)
        tm1 = jnp.concatenate([zero1, t[:-1]], axis=0)
        tp4 = jnp.concatenate([t[4:], zero4], axis=0)
        tm4 = jnp.concatenate([zero4, t[:-4]], axis=0)
        j = lax.broadcasted_iota(jnp.int32, (_NTOK, 1), 0)
        cpos = j % 4
        rpos = (j % 16) // 4
        s0 = jnp.where(cpos < 3, tp1, 0.0)
        s1 = jnp.where(cpos > 0, tm1, 0.0)
        s2 = jnp.where(rpos < 3, tp4, 0.0)
        s3 = jnp.where(rpos > 0, tm4, 0.0)
        ch = lax.broadcasted_iota(jnp.int32, (_NTOK, _G4), 1)
        s = jnp.where(ch < 24, s0,
                      jnp.where(ch < 48, s1, jnp.where(ch < 72, s2, s3)))
        su = jnp.dot(s.astype(jnp.bfloat16), wuT_ref[...],
                     preferred_element_type=jnp.float32)
        out_ref[b] = x_g + su + bu_ref[...]


def _make_sc_gather(total_rows, D, chunk):
    info = plsc.get_sparse_core_info()
    NC, NS_ = info.num_cores, info.num_subcores
    NW = NC * NS_
    per_w = total_rows // NW
    nch = per_w // chunk
    mesh = plsc.VectorSubcoreMesh(core_axis_name="c", subcore_axis_name="s")

    @functools.partial(
        pl.kernel, mesh=mesh,
        out_type=jax.ShapeDtypeStruct((total_rows, D), jnp.float32),
        scratch_types=[
            pltpu.VMEM((chunk,), jnp.int32),
            pltpu.VMEM((chunk, D), jnp.float32),
            pltpu.SemaphoreType.DMA,
        ],
    )
    def k(table_hbm, idx_hbm, out_hbm, idx_v, rows_v, sem):
        wid = lax.axis_index("s") * NC + lax.axis_index("c")
        for ci in range(nch):
            base = wid * per_w + ci * chunk
            pltpu.sync_copy(idx_hbm.at[pl.ds(base, chunk)], idx_v)
            pltpu.async_copy(table_hbm.at[idx_v], rows_v, sem).wait()
            pltpu.sync_copy(rows_v, out_hbm.at[pl.ds(base, chunk)])

    return k


def kernel(z, x, w_down, b_down, w_up, b_up):
    B, N_t, C = z.shape
    N_s = x.shape[1]

    n = lax.broadcasted_iota(jnp.int32, (_WIN, _NS), 1)
    w = lax.broadcasted_iota(jnp.int32, (_WIN, _NS), 0)
    pool = (((n // 128) == (w // 8)) & (((n % 32) // 4) == (w % 8))
            ).astype(jnp.bfloat16)

    rows = pl.pallas_call(
        _sel_body,
        grid=(B // _NB,),
        in_specs=[
            pl.BlockSpec((_NB, N_t, C), lambda b: (b, 0, 0)),
            pl.BlockSpec((_NB, N_s, C), lambda b: (b, 0, 0)),
            pl.BlockSpec((_WIN, _NS), lambda b: (0, 0)),
        ],
        out_specs=pl.BlockSpec((_NB, _NTOK), lambda b: (b, 0)),
        out_shape=jax.ShapeDtypeStruct((B, _NTOK), jnp.int32),
        compiler_params=pltpu.CompilerParams(
            dimension_semantics=("arbitrary",),
        ),
    )(z, x, pool)

    x2 = x.reshape(B * N_s, C)
    rows_flat = rows.reshape(B * _NTOK)
    xg2 = _make_sc_gather(B * _NTOK, C, 256)(x2, rows_flat)
    xg = xg2.reshape(B, _NTOK, C)

    out = pl.pallas_call(
        _proj_body,
        grid=(B // _NB,),
        in_specs=[
            pl.BlockSpec((_NB, _NTOK, C), lambda b: (b, 0, 0)),
            pl.BlockSpec((C, _G4), lambda b: (0, 0)),
            pl.BlockSpec((1, _G4), lambda b: (0, 0)),
            pl.BlockSpec((_G4, C), lambda b: (0, 0)),
            pl.BlockSpec((1, C), lambda b: (0, 0)),
        ],
        out_specs=pl.BlockSpec((_NB, _NTOK, C), lambda b: (b, 0, 0)),
        out_shape=jax.ShapeDtypeStruct((B, _NTOK, C), jnp.float32),
        compiler_params=pltpu.CompilerParams(
            dimension_semantics=("arbitrary",),
        ),
    )(xg, w_down.T.astype(jnp.bfloat16), b_down.reshape(1, -1),
      w_up.T.astype(jnp.bfloat16), b_up.reshape(1, -1))
    return out
